# baseline (speedup 1.0000x reference)
import numpy as np
import jax
import jax.numpy as jnp
from jax.experimental import pallas as pl
from jax.experimental.pallas import tpu as pltpu

_CONV_DTYPE = jnp.bfloat16
_BLOCK_B = 2048
_NEG = -1e30


def _round_up(v, m):
    return ((v + m - 1) // m) * m


def _make_sel():
    s = np.zeros((4, 10, 6, 24, 5, 5), np.float32)
    for w in range(4):
        dy, dx = divmod(w, 2)
        for ky in range(5):
            for kx in range(5):
                for j in range(10):
                    s[w, j, dy + ky, 2 * j + dx + kx, ky, kx] = 1.0
    return s


_SEL = _make_sel()


def _lenet_kernel(x_ref, w3_ref, bc_ref, wf_ref, bf_ref, o_ref):
    bc = bc_ref[...]

    hs = []
    y = None
    for i in range(10):
        w0 = min(max(2 * i - 2, 0), 14)
        v = 0 if i == 0 else (2 if i == 9 else 1)
        w3 = w3_ref[v]
        xr = x_ref[w0 * 24:w0 * 24 + 144, :]
        c0 = jnp.dot(w3[0:160], xr, preferred_element_type=jnp.float32)
        c1 = jnp.dot(w3[160:320], xr, preferred_element_type=jnp.float32)
        m01 = jnp.maximum(c0, c1).astype(jnp.bfloat16)
        c2 = jnp.dot(w3[320:480], xr, preferred_element_type=jnp.float32)
        c3 = jnp.dot(w3[480:640], xr, preferred_element_type=jnp.float32)
        m23 = jnp.maximum(c2, c3).astype(jnp.bfloat16)
        pooled = jnp.maximum(m01, m23)
        hs.append(jnp.maximum(pooled + bc, 0.0))
        if i == 4:
            h0 = jnp.concatenate(hs, axis=0)
            y = jnp.dot(wf_ref[0], h0,
                        preferred_element_type=jnp.float32)
            hs = []

    h1 = jnp.concatenate(hs, axis=0)
    y = y + jnp.dot(wf_ref[1], h1,
                    preferred_element_type=jnp.float32)
    y = y + bf_ref[...]

    m = jnp.max(y, axis=0, keepdims=True)
    z = y - m
    lse = jnp.log(jnp.sum(jnp.exp(z), axis=0, keepdims=True))
    o_ref[...] = z - lse


@jax.jit
def _forward(x, conv_w, conv_b, fc_w, fc_b):
    n = x.shape[0]
    npad = _round_up(n, _BLOCK_B)

    xp = jnp.pad(x.astype(_CONV_DTYPE), ((0, npad - n), (0, 0), (2, 2)))
    x2 = xp.reshape(npad, 480).T

    w3p = jnp.einsum("cab,wjrsab->wcjrs",
                     conv_w.reshape(16, 5, 5).astype(jnp.float32),
                     jnp.asarray(_SEL)).reshape(640, 6, 24)
    zero2 = jnp.zeros((640, 2, 24), jnp.float32)
    w_first = jnp.concatenate([w3p[:, 2:6], zero2], axis=1)
    w_last = jnp.concatenate([zero2, w3p[:, 0:4]], axis=1)
    w3 = jnp.stack([w_first, w3p, w_last])
    w3 = w3.reshape(3, 640, 144).astype(_CONV_DTYPE)

    bc2 = jnp.repeat(conv_b.astype(jnp.float32), 10).reshape(160, 1)
    bc2 = bc2.astype(jnp.bfloat16)

    wf2 = fc_w.astype(jnp.float32).reshape(10, 16, 10, 10)
    wf2 = wf2.transpose(0, 2, 1, 3).reshape(10, 1600)
    wf2 = jnp.pad(wf2, ((0, 6), (0, 0))).astype(jnp.bfloat16)
    wf2 = wf2.reshape(16, 2, 800).transpose(1, 0, 2)

    bf2 = jnp.full((16, 1), _NEG, jnp.float32).at[:10, 0].set(
        fc_b.astype(jnp.float32))

    grid = (npad // _BLOCK_B,)
    out = pl.pallas_call(
        _lenet_kernel,
        out_shape=jax.ShapeDtypeStruct((16, npad), jnp.float32),
        grid_spec=pltpu.PrefetchScalarGridSpec(
            num_scalar_prefetch=0,
            grid=grid,
            in_specs=[
                pl.BlockSpec((480, _BLOCK_B), lambda i: (0, i)),
                pl.BlockSpec((3, 640, 144), lambda i: (0, 0, 0)),
                pl.BlockSpec((160, 1), lambda i: (0, 0)),
                pl.BlockSpec((2, 16, 800), lambda i: (0, 0, 0)),
                pl.BlockSpec((16, 1), lambda i: (0, 0)),
            ],
            out_specs=pl.BlockSpec((16, _BLOCK_B), lambda i: (0, i)),
        ),
        compiler_params=pltpu.CompilerParams(
            dimension_semantics=("parallel",),
            vmem_limit_bytes=64 * 1024 * 1024,
        ),
    )(x2, w3, bc2, wf2, bf2)
    return out[:10, :n].T


def kernel(x, conv_w, conv_b, fc_w, fc_b):
    return _forward(x, conv_w, conv_b, fc_w, fc_b)


# speedup vs baseline: 1.0066x; 1.0066x over previous
import numpy as np
import jax
import jax.numpy as jnp
from jax.experimental import pallas as pl
from jax.experimental.pallas import tpu as pltpu

_CONV_DTYPE = jnp.bfloat16
_BLOCK_B = 4096
_NEG = -1e30


def _round_up(v, m):
    return ((v + m - 1) // m) * m


def _make_sel():
    s = np.zeros((4, 10, 6, 24, 5, 5), np.float32)
    for w in range(4):
        dy, dx = divmod(w, 2)
        for ky in range(5):
            for kx in range(5):
                for j in range(10):
                    s[w, j, dy + ky, 2 * j + dx + kx, ky, kx] = 1.0
    return s


_SEL = _make_sel()


def _lenet_kernel(x_ref, w3_ref, bc_ref, wf_ref, bf_ref, o_ref):
    bc = bc_ref[...]

    hs = []
    y = None
    for i in range(10):
        w0 = min(max(2 * i - 2, 0), 14)
        v = 0 if i == 0 else (2 if i == 9 else 1)
        w3 = w3_ref[v]
        xr = x_ref[w0 * 24:w0 * 24 + 144, :]
        c0 = jnp.dot(w3[0:160], xr, preferred_element_type=jnp.float32)
        c1 = jnp.dot(w3[160:320], xr, preferred_element_type=jnp.float32)
        m01 = jnp.maximum(c0, c1).astype(jnp.bfloat16)
        c2 = jnp.dot(w3[320:480], xr, preferred_element_type=jnp.float32)
        c3 = jnp.dot(w3[480:640], xr, preferred_element_type=jnp.float32)
        m23 = jnp.maximum(c2, c3).astype(jnp.bfloat16)
        pooled = jnp.maximum(m01, m23)
        hs.append(jnp.maximum(pooled + bc, 0.0))
        if i == 4:
            h0 = jnp.concatenate(hs, axis=0)
            y = jnp.dot(wf_ref[0], h0,
                        preferred_element_type=jnp.float32)
            hs = []

    h1 = jnp.concatenate(hs, axis=0)
    y = y + jnp.dot(wf_ref[1], h1,
                    preferred_element_type=jnp.float32)
    y = y + bf_ref[...]

    m = jnp.max(y, axis=0, keepdims=True)
    z = y - m
    lse = jnp.log(jnp.sum(jnp.exp(z), axis=0, keepdims=True))
    o_ref[...] = z - lse


@jax.jit
def _forward(x, conv_w, conv_b, fc_w, fc_b):
    n = x.shape[0]
    npad = _round_up(n, _BLOCK_B)

    xp = jnp.pad(x.astype(_CONV_DTYPE), ((0, npad - n), (0, 0), (2, 2)))
    x2 = xp.reshape(npad, 480).T

    w3p = jnp.einsum("cab,wjrsab->wcjrs",
                     conv_w.reshape(16, 5, 5).astype(jnp.float32),
                     jnp.asarray(_SEL)).reshape(640, 6, 24)
    zero2 = jnp.zeros((640, 2, 24), jnp.float32)
    w_first = jnp.concatenate([w3p[:, 2:6], zero2], axis=1)
    w_last = jnp.concatenate([zero2, w3p[:, 0:4]], axis=1)
    w3 = jnp.stack([w_first, w3p, w_last])
    w3 = w3.reshape(3, 640, 144).astype(_CONV_DTYPE)

    bc2 = jnp.repeat(conv_b.astype(jnp.float32), 10).reshape(160, 1)
    bc2 = bc2.astype(jnp.bfloat16)

    wf2 = fc_w.astype(jnp.float32).reshape(10, 16, 10, 10)
    wf2 = wf2.transpose(0, 2, 1, 3).reshape(10, 1600)
    wf2 = jnp.pad(wf2, ((0, 6), (0, 0))).astype(jnp.bfloat16)
    wf2 = wf2.reshape(16, 2, 800).transpose(1, 0, 2)

    bf2 = jnp.full((16, 1), _NEG, jnp.float32).at[:10, 0].set(
        fc_b.astype(jnp.float32))

    grid = (npad // _BLOCK_B,)
    out = pl.pallas_call(
        _lenet_kernel,
        out_shape=jax.ShapeDtypeStruct((16, npad), jnp.float32),
        grid_spec=pltpu.PrefetchScalarGridSpec(
            num_scalar_prefetch=0,
            grid=grid,
            in_specs=[
                pl.BlockSpec((480, _BLOCK_B), lambda i: (0, i)),
                pl.BlockSpec((3, 640, 144), lambda i: (0, 0, 0)),
                pl.BlockSpec((160, 1), lambda i: (0, 0)),
                pl.BlockSpec((2, 16, 800), lambda i: (0, 0, 0)),
                pl.BlockSpec((16, 1), lambda i: (0, 0)),
            ],
            out_specs=pl.BlockSpec((16, _BLOCK_B), lambda i: (0, i)),
        ),
        compiler_params=pltpu.CompilerParams(
            dimension_semantics=("parallel",),
            vmem_limit_bytes=64 * 1024 * 1024,
        ),
    )(x2, w3, bc2, wf2, bf2)
    return out[:10, :n].T


def kernel(x, conv_w, conv_b, fc_w, fc_b):
    return _forward(x, conv_w, conv_b, fc_w, fc_b)
